# baseline (speedup 1.0000x reference)
import jax
import jax.numpy as jnp
from jax.experimental import pallas as pl
from jax.experimental.pallas import tpu as pltpu

_NUM_CLASSES = 80
_IGNORE_THRESH = 0.5
_ROWS = 4


def _make_body(B, T, G, A, attrs, R):
    NB = T * B

    def _body(tp_ref, tl_ref, anc_ref, out_ref,
              z0_scr, z1_scr, z2_scr, c0_scr, c1_scr, c2_scr):
        j = pl.program_id(0)
        x = tp_ref[...]
        scrs = (z0_scr, z1_scr, z2_scr)
        cnts = (c0_scr, c1_scr, c2_scr)
        for r in range(R):
            xr = x[r]
            for a in range(A):
                c = a * attrs + 4
                za = xr[:, :, c:c + 1].reshape(G, B)
                row = (j * R + r) * B
                scrs[a][pl.ds(row, B), :] = za.T

        @pl.when(j == 0)
        def _mask():
            t = tl_ref[...]
            t0, t1, t2, t3, t4 = t[0], t[1], t[2], t[3], t[4]
            valid = (t0 + t1 + t2 + t3 + t4) != 0.0
            gx = t1 * G
            gy = t2 * G
            gw = t3 * G
            gh = t4 * G
            gi = gx.astype(jnp.int32)
            gj = gy.astype(jnp.int32)

            ious = []
            for a in range(A):
                aw = anc_ref[a, 0]
                ah = anc_ref[a, 1]
                iw = jnp.clip(jnp.minimum(gw / 2, aw / 2) - jnp.maximum(-gw / 2, -aw / 2) + 1.0, 0.0, None)
                ih = jnp.clip(jnp.minimum(gh / 2, ah / 2) - jnp.maximum(-gh / 2, -ah / 2) + 1.0, 0.0, None)
                inter = iw * ih
                a1 = (gw + 1.0) * (gh + 1.0)
                a2 = (aw + 1.0) * (ah + 1.0)
                ious.append(inter / (a1 + a2 - inter + 1e-16))
            i0, i1, i2 = ious
            b01 = i1 > i0
            best_iou = jnp.where(b01, i1, i0)
            best_n = jnp.where(b01, 1, 0)
            b2 = i2 > best_iou
            best_iou = jnp.where(b2, i2, best_iou)
            best_n = jnp.where(b2, 2, best_n)
            cond_ign = valid & (best_iou > _IGNORE_THRESH)

            b_idx = jax.lax.broadcasted_iota(jnp.int32, (1, NB), 1) // T
            rkey = gj * B + b_idx

            col_iota = jax.lax.broadcasted_iota(jnp.int32, (G, NB), 0)
            u2 = jnp.where(gi == col_iota, 1.0, 0.0)

            n_ch = 4
            rows = G * B // n_ch
            base_iota = jax.lax.broadcasted_iota(jnp.int32, (rows, NB), 0)
            for a in range(A):
                key_a = jnp.where(cond_ign & (best_n == a), rkey, -1)
                for ch in range(n_ch):
                    u1 = jnp.where(key_a == base_iota + ch * rows, 1.0, 0.0)
                    cnts[a][ch * rows:(ch + 1) * rows, :] = jax.lax.dot_general(
                        u1, u2,
                        dimension_numbers=(((1,), (1,)), ((), ())),
                        preferred_element_type=jnp.float32,
                    )

        @pl.when(j == G // R - 1)
        def _finish():
            total = jnp.float32(0.0)
            for a in range(A):
                z = scrs[a][...]
                s = jax.nn.sigmoid(z)
                f = -jnp.maximum(jnp.log(1.0 - s), -100.0)
                total = total + jnp.sum(jnp.where(cnts[a][...] < 0.5, f, 0.0))
            out_ref[0, 0] = total
    return _body


def kernel(pred, target, anchors, num_anchors, grid_size):
    B, C, G, _ = pred.shape
    A = anchors.shape[0]
    T = target.shape[1]
    attrs = C // A
    R = _ROWS if G % _ROWS == 0 else 1
    scaled_anchors = (anchors / (grid_size // G)) * (num_anchors // A)

    tp = jnp.transpose(pred, (2, 3, 0, 1))
    tl = jnp.transpose(target, (2, 0, 1)).reshape(5, 1, B * T)

    out = pl.pallas_call(
        _make_body(B, T, G, A, attrs, R),
        grid=(G // R,),
        out_shape=jax.ShapeDtypeStruct((1, 1), jnp.float32),
        in_specs=[
            pl.BlockSpec((R, G, B, C), lambda j: (j, 0, 0, 0)),
            pl.BlockSpec(tl.shape, lambda j: (0, 0, 0)),
            pl.BlockSpec(memory_space=pltpu.SMEM),
        ],
        out_specs=pl.BlockSpec(memory_space=pltpu.SMEM),
        scratch_shapes=[pltpu.VMEM((G * B, G), jnp.float32)] * (2 * A),
    )(tp, tl, scaled_anchors)
    return out[0, 0]
